# baseline (speedup 1.0000x reference)
import functools

import jax
import jax.numpy as jnp
from jax import lax
from jax.experimental import pallas as pl
from jax.experimental.pallas import tpu as pltpu
from jax.experimental.pallas import tpu_sc as plsc

BATCH = 16384
DIM = 128
CHUNK = 128


def _body(types_hbm, labels_hbm, table_hbm, out_hbm,
          t_v, l_v, idx_v, rows_v, gsem):
    info = plsc.get_sparse_core_info()
    nc, lanes = info.num_cores, info.num_lanes
    bpw = BATCH // (nc * info.num_subcores)
    nchunk = bpw // CHUNK

    wid = lax.axis_index("s") * nc + lax.axis_index("c")
    base = wid * bpw

    pltpu.sync_copy(types_hbm.at[pl.ds(base, bpw)], t_v)
    pltpu.sync_copy(labels_hbm.at[pl.ds(base, bpw)], l_v)

    for j in range(nchunk):
        for i in range(CHUNK // lanes):
            s = pl.ds(j * CHUNK + i * lanes, lanes)
            idx_v[j, pl.ds(i * lanes, lanes)] = t_v[s] * 2 + l_v[s]

    copies = [
        pltpu.async_copy(table_hbm.at[idx_v.at[j]],
                         rows_v.at[pl.ds(j * CHUNK, CHUNK)],
                         gsem.at[j])
        for j in range(nchunk)
    ]
    for c in copies:
        c.wait()
    pltpu.sync_copy(rows_v, out_hbm.at[pl.ds(base, bpw)])


def kernel(interaction_types, labels, embedding_weight):
    info = plsc.get_sparse_core_info()
    nw = info.num_cores * info.num_subcores
    bpw = BATCH // nw
    nchunk = bpw // CHUNK
    mesh = plsc.VectorSubcoreMesh(core_axis_name="c", subcore_axis_name="s")

    f = functools.partial(
        pl.kernel,
        mesh=mesh,
        out_type=jax.ShapeDtypeStruct((BATCH, DIM), jnp.float32),
        scratch_types=[
            pltpu.VMEM((bpw,), jnp.int32),
            pltpu.VMEM((bpw,), jnp.int32),
            pltpu.VMEM((nchunk, CHUNK), jnp.int32),
            pltpu.VMEM((bpw, DIM), jnp.float32),
            pltpu.SemaphoreType.DMA((nchunk,)),
        ],
    )(_body)
    return f(interaction_types.astype(jnp.int32),
             labels.astype(jnp.int32),
             embedding_weight)


# speedup vs baseline: 1.1023x; 1.1023x over previous
import functools

import jax
import jax.numpy as jnp
from jax import lax
from jax.experimental import pallas as pl
from jax.experimental.pallas import tpu as pltpu
from jax.experimental.pallas import tpu_sc as plsc

BATCH = 16384
DIM = 128
NROWS = 8
OUT_CHUNKS = 4


def _body(types_hbm, labels_hbm, table_hbm, out_hbm,
          t_v, l_v, idx_v, table_v, out_v, osem):
    info = plsc.get_sparse_core_info()
    nc, lanes = info.num_cores, info.num_lanes
    bpw = BATCH // (nc * info.num_subcores)
    groups = bpw // lanes

    wid = lax.axis_index("s") * nc + lax.axis_index("c")
    base = wid * bpw

    pltpu.sync_copy(types_hbm.at[pl.ds(base, bpw)], t_v)
    pltpu.sync_copy(labels_hbm.at[pl.ds(base, bpw)], l_v)
    pltpu.sync_copy(table_hbm, table_v)

    iota = lax.iota(jnp.int32, lanes)

    for g in range(groups):
        s = pl.ds(g * lanes, lanes)
        idx_v[s] = t_v[s] * 2 + l_v[s]

    def do_group(g, carry):
        ids128 = idx_v[pl.ds(g * lanes, lanes)] * DIM
        b128 = (iota + g * lanes) * DIM
        for d in range(DIM):
            d_splat = jnp.full((lanes,), d, jnp.int32)
            vals = plsc.load_gather(table_v, [ids128 + d_splat])
            plsc.store_scatter(out_v, [b128 + d_splat], vals)
        return carry

    gpc = groups // OUT_CHUNKS
    elems = gpc * lanes * DIM
    copies = []
    for c in range(OUT_CHUNKS):
        lax.fori_loop(c * gpc, (c + 1) * gpc, do_group, 0, unroll=False)
        src = pl.ds(c * elems, elems)
        dst = pl.ds(base * DIM + c * elems, elems)
        copies.append(pltpu.async_copy(out_v.at[src], out_hbm.at[dst], osem))
    for cp in copies:
        cp.wait()


def kernel(interaction_types, labels, embedding_weight):
    info = plsc.get_sparse_core_info()
    nw = info.num_cores * info.num_subcores
    bpw = BATCH // nw
    mesh = plsc.VectorSubcoreMesh(core_axis_name="c", subcore_axis_name="s")

    f = functools.partial(
        pl.kernel,
        mesh=mesh,
        compiler_params=pltpu.CompilerParams(needs_layout_passes=False),
        out_type=jax.ShapeDtypeStruct((BATCH * DIM,), jnp.float32),
        scratch_types=[
            pltpu.VMEM((bpw,), jnp.int32),
            pltpu.VMEM((bpw,), jnp.int32),
            pltpu.VMEM((bpw,), jnp.int32),
            pltpu.VMEM((NROWS * DIM,), jnp.float32),
            pltpu.VMEM((bpw * DIM,), jnp.float32),
            pltpu.SemaphoreType.DMA,
        ],
    )(_body)
    out = f(interaction_types.astype(jnp.int32),
            labels.astype(jnp.int32),
            embedding_weight.reshape(NROWS * DIM))
    return out.reshape(BATCH, DIM)


# speedup vs baseline: 3.0511x; 2.7680x over previous
import functools

import jax
import jax.numpy as jnp
from jax import lax
from jax.experimental import pallas as pl
from jax.experimental.pallas import tpu as pltpu
from jax.experimental.pallas import tpu_sc as plsc

BATCH = 16384
DIM = 128
NROWS = 8
CHUNK = 128


def _body(types_hbm, labels_hbm, table_hbm, out_hbm,
          t_v, l_v, idx_v, rows_v, gsem, osem):
    info = plsc.get_sparse_core_info()
    nc, lanes = info.num_cores, info.num_lanes
    bpw = BATCH // (nc * info.num_subcores)
    nchunk = bpw // CHUNK

    wid = lax.axis_index("s") * nc + lax.axis_index("c")
    base = wid * bpw
    row_off = wid * NROWS

    pltpu.sync_copy(types_hbm.at[pl.ds(base, bpw)], t_v)
    pltpu.sync_copy(labels_hbm.at[pl.ds(base, bpw)], l_v)

    for j in range(nchunk):
        for i in range(CHUNK // lanes):
            s = pl.ds(j * CHUNK + i * lanes, lanes)
            idx_v[j, pl.ds(i * lanes, lanes)] = t_v[s] * 2 + l_v[s] + row_off

    gathers = [
        pltpu.async_copy(table_hbm.at[idx_v.at[j]],
                         rows_v.at[pl.ds(j * CHUNK, CHUNK)],
                         gsem.at[j])
        for j in range(nchunk)
    ]
    stores = []
    for j in range(nchunk):
        gathers[j].wait()
        stores.append(
            pltpu.async_copy(rows_v.at[pl.ds(j * CHUNK, CHUNK)],
                             out_hbm.at[pl.ds(base + j * CHUNK, CHUNK)],
                             osem))
    for s in stores:
        s.wait()


def kernel(interaction_types, labels, embedding_weight):
    info = plsc.get_sparse_core_info()
    nw = info.num_cores * info.num_subcores
    bpw = BATCH // nw
    nchunk = bpw // CHUNK
    mesh = plsc.VectorSubcoreMesh(core_axis_name="c", subcore_axis_name="s")

    f = functools.partial(
        pl.kernel,
        mesh=mesh,
        out_type=jax.ShapeDtypeStruct((BATCH, DIM), jnp.float32),
        scratch_types=[
            pltpu.VMEM((bpw,), jnp.int32),
            pltpu.VMEM((bpw,), jnp.int32),
            pltpu.VMEM((nchunk, CHUNK), jnp.int32),
            pltpu.VMEM((bpw, DIM), jnp.float32),
            pltpu.SemaphoreType.DMA((nchunk,)),
            pltpu.SemaphoreType.DMA,
        ],
    )(_body)
    rep_table = jnp.tile(embedding_weight, (nw, 1))
    return f(interaction_types.astype(jnp.int32),
             labels.astype(jnp.int32),
             rep_table)


# speedup vs baseline: 3.1913x; 1.0460x over previous
import functools

import jax
import jax.numpy as jnp
from jax import lax
from jax.experimental import pallas as pl
from jax.experimental.pallas import tpu as pltpu
from jax.experimental.pallas import tpu_sc as plsc

BATCH = 16384
DIM = 128
NROWS = 8
CHUNK = 128


def _body(types_hbm, labels_hbm, table_hbm, out_hbm, rep_hbm,
          t_v, l_v, idx_v, table_v, rows_v, gsem, osem):
    info = plsc.get_sparse_core_info()
    nc, lanes = info.num_cores, info.num_lanes
    bpw = BATCH // (nc * info.num_subcores)
    nchunk = bpw // CHUNK

    wid = lax.axis_index("s") * nc + lax.axis_index("c")
    base = wid * bpw
    row_off = wid * NROWS

    tload = pltpu.async_copy(table_hbm, table_v, gsem.at[0])
    pltpu.sync_copy(types_hbm.at[pl.ds(base, bpw)], t_v)
    pltpu.sync_copy(labels_hbm.at[pl.ds(base, bpw)], l_v)

    for j in range(nchunk):
        for i in range(CHUNK // lanes):
            s = pl.ds(j * CHUNK + i * lanes, lanes)
            idx_v[j, pl.ds(i * lanes, lanes)] = t_v[s] * 2 + l_v[s] + row_off

    tload.wait()
    pltpu.sync_copy(table_v, rep_hbm.at[pl.ds(row_off, NROWS)])

    gathers = [
        pltpu.async_copy(rep_hbm.at[idx_v.at[j]],
                         rows_v.at[pl.ds(j * CHUNK, CHUNK)],
                         gsem.at[j])
        for j in range(nchunk)
    ]
    stores = []
    for j in range(nchunk):
        gathers[j].wait()
        stores.append(
            pltpu.async_copy(rows_v.at[pl.ds(j * CHUNK, CHUNK)],
                             out_hbm.at[pl.ds(base + j * CHUNK, CHUNK)],
                             osem))
    for s in stores:
        s.wait()


def kernel(interaction_types, labels, embedding_weight):
    info = plsc.get_sparse_core_info()
    nw = info.num_cores * info.num_subcores
    bpw = BATCH // nw
    nchunk = bpw // CHUNK
    mesh = plsc.VectorSubcoreMesh(core_axis_name="c", subcore_axis_name="s")

    f = functools.partial(
        pl.kernel,
        mesh=mesh,
        out_type=(
            jax.ShapeDtypeStruct((BATCH, DIM), jnp.float32),
            jax.ShapeDtypeStruct((nw * NROWS, DIM), jnp.float32),
        ),
        scratch_types=[
            pltpu.VMEM((bpw,), jnp.int32),
            pltpu.VMEM((bpw,), jnp.int32),
            pltpu.VMEM((nchunk, CHUNK), jnp.int32),
            pltpu.VMEM((NROWS, DIM), jnp.float32),
            pltpu.VMEM((bpw, DIM), jnp.float32),
            pltpu.SemaphoreType.DMA((nchunk,)),
            pltpu.SemaphoreType.DMA,
        ],
    )(_body)
    out, _ = f(interaction_types.astype(jnp.int32),
               labels.astype(jnp.int32),
               embedding_weight)
    return out


# speedup vs baseline: 4.4081x; 1.3813x over previous
import functools

import jax
import jax.numpy as jnp
from jax import lax
from jax.experimental import pallas as pl
from jax.experimental.pallas import tpu as pltpu
from jax.experimental.pallas import tpu_sc as plsc

BATCH = 16384
DIM = 128
NROWS = 8
CHUNK = 128


def _body(types_hbm, labels_hbm, table_hbm, out_hbm,
          t_v, l_v, idx_v, table_v, stab, rows_v, gsem, osem):
    info = plsc.get_sparse_core_info()
    nc, lanes = info.num_cores, info.num_lanes
    bpw = BATCH // (nc * info.num_subcores)
    nchunk = bpw // CHUNK

    sid = lax.axis_index("s")
    wid = sid * nc + lax.axis_index("c")
    base = wid * bpw
    row_off = sid * NROWS

    tload = pltpu.async_copy(table_hbm, table_v, gsem.at[0])
    pltpu.sync_copy(types_hbm.at[pl.ds(base, bpw)], t_v)
    pltpu.sync_copy(labels_hbm.at[pl.ds(base, bpw)], l_v)

    for j in range(nchunk):
        for i in range(CHUNK // lanes):
            s = pl.ds(j * CHUNK + i * lanes, lanes)
            idx_v[j, pl.ds(i * lanes, lanes)] = t_v[s] * 2 + l_v[s] + row_off
    tload.wait()
    pltpu.sync_copy(table_v, stab.at[pl.ds(row_off, NROWS)])

    gathers = [
        pltpu.async_copy(stab.at[idx_v.at[j]],
                         rows_v.at[pl.ds(j * CHUNK, CHUNK)],
                         gsem.at[j])
        for j in range(nchunk)
    ]
    stores = []
    for j in range(nchunk):
        gathers[j].wait()
        stores.append(
            pltpu.async_copy(rows_v.at[pl.ds(j * CHUNK, CHUNK)],
                             out_hbm.at[pl.ds(base + j * CHUNK, CHUNK)],
                             osem))
    for s in stores:
        s.wait()


def kernel(interaction_types, labels, embedding_weight):
    info = plsc.get_sparse_core_info()
    nw = info.num_cores * info.num_subcores
    bpw = BATCH // nw
    nchunk = bpw // CHUNK
    mesh = plsc.VectorSubcoreMesh(core_axis_name="c", subcore_axis_name="s")

    f = functools.partial(
        pl.kernel,
        mesh=mesh,
        out_type=jax.ShapeDtypeStruct((BATCH, DIM), jnp.float32),
        scratch_types=[
            pltpu.VMEM((bpw,), jnp.int32),
            pltpu.VMEM((bpw,), jnp.int32),
            pltpu.VMEM((nchunk, CHUNK), jnp.int32),
            pltpu.VMEM((NROWS, DIM), jnp.float32),
            pltpu.VMEM_SHARED((16 * NROWS, DIM), jnp.float32),
            pltpu.VMEM((bpw, DIM), jnp.float32),
            pltpu.SemaphoreType.DMA((nchunk,)),
            pltpu.SemaphoreType.DMA,
        ],
    )(_body)
    return f(interaction_types.astype(jnp.int32),
             labels.astype(jnp.int32),
             embedding_weight)


# speedup vs baseline: 4.4098x; 1.0004x over previous
import functools

import jax
import jax.numpy as jnp
from jax import lax
from jax.experimental import pallas as pl
from jax.experimental.pallas import tpu as pltpu
from jax.experimental.pallas import tpu_sc as plsc

BATCH = 16384
DIM = 128
NROWS = 8
CHUNK = 128


def _body(types_hbm, labels_hbm, table_hbm, out_hbm,
          t_v, l_v, idx_v, table_v, stab, rows_v, gsem, osem):
    info = plsc.get_sparse_core_info()
    nc, lanes = info.num_cores, info.num_lanes
    bpw = BATCH // (nc * info.num_subcores)
    nchunk = bpw // CHUNK

    sid = lax.axis_index("s")
    wid = sid * nc + lax.axis_index("c")
    base = wid * bpw
    row_off = sid * NROWS

    tload = pltpu.async_copy(table_hbm, table_v, gsem.at[0])
    pltpu.sync_copy(types_hbm.at[pl.ds(base, bpw)], t_v)
    pltpu.sync_copy(labels_hbm.at[pl.ds(base, bpw)], l_v)

    for j in range(nchunk):
        for i in range(CHUNK // lanes):
            s = pl.ds(j * CHUNK + i * lanes, lanes)
            idx_v[j, pl.ds(i * lanes, lanes)] = t_v[s] * 2 + l_v[s] + row_off
    tload.wait()
    pltpu.sync_copy(table_v, stab.at[pl.ds(row_off, NROWS)])

    gathers = [
        pltpu.async_copy(stab.at[idx_v.at[j]],
                         rows_v.at[pl.ds(j * CHUNK, CHUNK)],
                         gsem.at[j])
        for j in range(nchunk)
    ]
    stores = []
    for j in range(nchunk):
        gathers[j].wait()
        stores.append(
            pltpu.async_copy(rows_v.at[pl.ds(j * CHUNK, CHUNK)],
                             out_hbm.at[pl.ds(base + j * CHUNK, CHUNK)],
                             osem))
    for s in stores:
        s.wait()


def kernel(interaction_types, labels, embedding_weight):
    info = plsc.get_sparse_core_info()
    nw = info.num_cores * info.num_subcores
    bpw = BATCH // nw
    nchunk = bpw // CHUNK
    mesh = plsc.VectorSubcoreMesh(core_axis_name="c", subcore_axis_name="s")

    f = functools.partial(
        pl.kernel,
        mesh=mesh,
        compiler_params=pltpu.CompilerParams(
            skip_device_barrier=True,
            disable_bounds_checks=True,
            disable_semaphore_checks=True,
        ),
        out_type=jax.ShapeDtypeStruct((BATCH, DIM), jnp.float32),
        scratch_types=[
            pltpu.VMEM((bpw,), jnp.int32),
            pltpu.VMEM((bpw,), jnp.int32),
            pltpu.VMEM((nchunk, CHUNK), jnp.int32),
            pltpu.VMEM((NROWS, DIM), jnp.float32),
            pltpu.VMEM_SHARED((16 * NROWS, DIM), jnp.float32),
            pltpu.VMEM((bpw, DIM), jnp.float32),
            pltpu.SemaphoreType.DMA((nchunk,)),
            pltpu.SemaphoreType.DMA,
        ],
    )(_body)
    return f(interaction_types.astype(jnp.int32),
             labels.astype(jnp.int32),
             embedding_weight)
